# baseline (speedup 1.0000x reference)
import functools

import jax
import jax.numpy as jnp
from jax import lax
from jax.experimental import pallas as pl
from jax.experimental.pallas import tpu as pltpu
from jax.experimental.pallas import tpu_sc as plsc

N_NODE = 10000
E = 320000
NC = 2
NS = 16
NW = NC * NS
EPW = E // NW
CHUNK = 80
RPT = EPW // CHUNK
ROWS = E // CHUNK
NPT = N_NODE // NS
NZB = NPT // 5

_mesh = plsc.VectorSubcoreMesh(core_axis_name="c", subcore_axis_name="s")
_sc_params = pltpu.CompilerParams(needs_layout_passes=False,
                                  use_tc_tiling_on_sc=False)


@functools.partial(
    pl.kernel, mesh=_mesh, compiler_params=_sc_params,
    out_type=jax.ShapeDtypeStruct((4, NC, N_NODE), jnp.float32),
    scratch_types=[
        pltpu.VMEM((RPT, CHUNK), jnp.int32),
        pltpu.VMEM((CHUNK,), jnp.float32),
        pltpu.VMEM((N_NODE,), jnp.float32),
        pltpu.VMEM_SHARED((N_NODE,), jnp.float32),
        pltpu.VMEM_SHARED((N_NODE,), jnp.float32),
        pltpu.VMEM_SHARED((N_NODE,), jnp.float32),
        pltpu.VMEM_SHARED((N_NODE,), jnp.float32),
        pltpu.SemaphoreType.DMA,
    ],
)
def _degree_hist(src_it, dst_it, src_ti, dst_ti, out, idxbuf, ones, zbuf,
                 a0, a1, a2, a3, sem):
    c = lax.axis_index("c")
    s = lax.axis_index("s")
    wid = s * NC + c
    z16 = jnp.zeros((16,), jnp.float32)
    one16 = jnp.ones((16,), jnp.float32)
    accs = (a0, a1, a2, a3)

    for i in range(CHUNK // 16):
        ones[pl.ds(i * 16, 16)] = one16

    @pl.when(s == 0)
    def _zero():
        def zbody(i, _):
            zbuf[pl.ds(i * 16, 16)] = z16
            return 0
        lax.fori_loop(0, N_NODE // 16, zbody, 0)
        for a in accs:
            pltpu.sync_copy(zbuf, a)
    plsc.subcore_barrier()

    for k, edges in enumerate((src_it, dst_it, src_ti, dst_ti)):
        a = accs[k]
        pltpu.sync_copy(edges.at[pl.ds(wid * RPT, RPT)], idxbuf)

        def fire(j, _):
            pltpu.async_copy(ones, a.at[idxbuf.at[j]], sem, add=True)
            return 0
        lax.fori_loop(0, RPT, fire, 0)

        def drain(j, _):
            pltpu.make_async_copy(ones, a.at[idxbuf.at[0]], sem).wait()
            return 0
        lax.fori_loop(0, RPT, drain, 0)
    plsc.subcore_barrier()

    @pl.when(s == 0)
    def _copyout():
        for k, a in enumerate(accs):
            pltpu.sync_copy(a, zbuf)
            pltpu.sync_copy(zbuf, out.at[k].at[c])


def _zero_acc(zbuf, acc, s, d):
    z16 = jnp.zeros((16,), jnp.float32)

    def zbody(r, _):
        for kcol in range(d // 16):
            zbuf[r, pl.ds(kcol * 16, 16)] = z16
        return 0
    lax.fori_loop(0, NZB, zbody, 0)
    for t in range(5):
        pltpu.sync_copy(zbuf, acc.at[pl.ds(s * NPT + t * NZB, NZB)])


def _copy_out(zbuf, acc, out, s, base):
    for t in range(5):
        off = s * NPT + t * NZB
        pltpu.sync_copy(acc.at[pl.ds(off, NZB)], zbuf)
        pltpu.sync_copy(zbuf, out.at[pl.ds(base + off, NZB)])


IB = 50


def _edge_loop(x, src2, dst2, base, rpt, idx_s, idx_d, rows0, rows1, acc,
               sem0, sem1):
    def blk(bi, _):
        pltpu.sync_copy(src2.at[pl.ds(base + bi * IB, IB)], idx_s)
        pltpu.sync_copy(dst2.at[pl.ds(base + bi * IB, IB)], idx_d)

        def ebody(j, _):
            pltpu.async_copy(x.at[idx_s.at[j]], rows0, sem0).wait()
            pltpu.sync_copy(rows0, acc.at[idx_d.at[j]], add=True)
            return 0
        lax.fori_loop(0, IB, ebody, 0)
        return 0
    lax.fori_loop(0, rpt // IB, blk, 0)


RPT2 = ROWS // NS


def _make_agg_pair(d):
    @functools.partial(
        pl.kernel, mesh=_mesh, compiler_params=_sc_params,
        out_type=jax.ShapeDtypeStruct((NC * N_NODE, d), jnp.float32),
        scratch_types=[
            pltpu.VMEM((IB, CHUNK), jnp.int32),
            pltpu.VMEM((IB, CHUNK), jnp.int32),
            pltpu.VMEM((CHUNK, d), jnp.float32),
            pltpu.VMEM((CHUNK, d), jnp.float32),
            pltpu.VMEM((NZB, d), jnp.float32),
            pltpu.VMEM_SHARED((N_NODE, d), jnp.float32),
            pltpu.SemaphoreType.DMA,
            pltpu.SemaphoreType.DMA,
        ],
    )
    def _agg(xa, xb, sa2, da2, sb2, db2, out, idx_s, idx_d, rows0, rows1,
             zbuf, acc, sem0, sem1):
        c = lax.axis_index("c")
        s = lax.axis_index("s")
        _zero_acc(zbuf, acc, s, d)
        plsc.subcore_barrier()

        @pl.when(c == 0)
        def _ca():
            _edge_loop(xa, sa2, da2, s * RPT2, RPT2, idx_s, idx_d, rows0,
                       rows1, acc, sem0, sem1)

        @pl.when(c == 1)
        def _cb():
            _edge_loop(xb, sb2, db2, s * RPT2, RPT2, idx_s, idx_d, rows0,
                       rows1, acc, sem0, sem1)
        plsc.subcore_barrier()
        _copy_out(zbuf, acc, out, s, c * N_NODE)

    return _agg


_agg_pair128 = _make_agg_pair(128)


BM = 2000


def _dot(a, b):
    return jnp.dot(a, b, preferred_element_type=jnp.float32)


def _scales_body(hp_ref, o_ref):
    deg = jnp.maximum(hp_ref[:, 0] + hp_ref[:, 1], 1.0)
    r = lax.rsqrt(deg)
    r = r * (1.5 - 0.5 * deg * r * r)
    r = r * (1.5 - 0.5 * deg * r * r)
    o_ref[...] = r


def _scales(hp):
    return pl.pallas_call(
        _scales_body,
        out_shape=jax.ShapeDtypeStruct((4, N_NODE), jnp.float32),
    )(hp)


def _rowscale_body(x_ref, s_ref, o_ref):
    o_ref[...] = x_ref[...] * s_ref[...]


def _rowscale(x, sc):
    d = x.shape[-1]
    return pl.pallas_call(
        _rowscale_body,
        grid=(N_NODE // BM,),
        in_specs=[pl.BlockSpec((BM, d), lambda i: (i, 0)),
                  pl.BlockSpec((BM, 1), lambda i: (i, 0))],
        out_specs=pl.BlockSpec((BM, d), lambda i: (i, 0)),
        out_shape=jax.ShapeDtypeStruct((N_NODE, d), jnp.float32),
    )(x, sc)


def _conv_post_body(relu, has_so, p_ref, si_ref, w_ref, b_ref, *rest):
    if has_so:
        so_ref, o_ref = rest
    else:
        (o_ref,) = rest
    v = _dot(p_ref[...] * si_ref[...], w_ref[...]) + b_ref[...]
    if relu:
        v = jnp.maximum(v, 0.0)
    if has_so:
        v = v * so_ref[...]
    o_ref[...] = v


def _conv_post(p, si, w, b, relu, so=None):
    din, dout = w.shape
    has_so = so is not None
    in_specs = [pl.BlockSpec((BM, din), lambda i: (i, 0)),
                pl.BlockSpec((BM, 1), lambda i: (i, 0)),
                pl.BlockSpec((din, dout), lambda i: (0, 0)),
                pl.BlockSpec((1, dout), lambda i: (0, 0))]
    args = [p, si, w, b.reshape(1, -1)]
    if has_so:
        in_specs.append(pl.BlockSpec((BM, 1), lambda i: (i, 0)))
        args.append(so)
    return pl.pallas_call(
        functools.partial(_conv_post_body, relu, has_so),
        grid=(N_NODE // BM,),
        in_specs=in_specs,
        out_specs=pl.BlockSpec((BM, dout), lambda i: (i, 0)),
        out_shape=jax.ShapeDtypeStruct((N_NODE, dout), jnp.float32),
    )(*args)


def _ff_body(x_ref, wi, bi, wh, bh, wo, bo, o_ref):
    h = jnp.maximum(_dot(x_ref[...], wi[...]) + bi[...], 0.0)
    h = jnp.maximum(_dot(h, wh[...]) + bh[...], 0.0)
    o_ref[...] = _dot(h, wo[...]) + bo[...]


def _ff(x, wi, bi, wh, bh, wo, bo):
    m, din = x.shape
    dh = wi.shape[1]
    dout = wo.shape[1]
    return pl.pallas_call(
        _ff_body,
        grid=(m // BM,),
        in_specs=[pl.BlockSpec((BM, din), lambda i: (i, 0)),
                  pl.BlockSpec((din, dh), lambda i: (0, 0)),
                  pl.BlockSpec((1, dh), lambda i: (0, 0)),
                  pl.BlockSpec((dh, dh), lambda i: (0, 0)),
                  pl.BlockSpec((1, dh), lambda i: (0, 0)),
                  pl.BlockSpec((dh, dout), lambda i: (0, 0)),
                  pl.BlockSpec((1, dout), lambda i: (0, 0))],
        out_specs=pl.BlockSpec((BM, dout), lambda i: (i, 0)),
        out_shape=jax.ShapeDtypeStruct((m, dout), jnp.float32),
    )(x, wi, bi.reshape(1, -1), wh, bh.reshape(1, -1), wo, bo.reshape(1, -1))


def kernel(input_features, edge_i2t, edge_t2i, embed_item,
           pre_Wi, pre_bi, pre_Wh, pre_bh, pre_Wo, pre_bo,
           c1_W_i2t, c1_b_i2t, c1_W_t2i, c1_b_t2i,
           c2_W_i2t, c2_b_i2t, c2_W_t2i, c2_b_t2i,
           c3_W_i2t, c3_b_i2t, c3_W_t2i, c3_b_t2i,
           post_Wi, post_bi, post_Wh, post_bh, post_Wo, post_bo):
    src_it = edge_i2t[0].astype(jnp.int32)
    dst_it = edge_i2t[1].astype(jnp.int32)
    src_ti = edge_t2i[0].astype(jnp.int32)
    dst_ti = edge_t2i[1].astype(jnp.int32)

    it_s2 = src_it.reshape(ROWS, CHUNK)
    it_d2 = dst_it.reshape(ROWS, CHUNK)
    ti_s2 = src_ti.reshape(ROWS, CHUNK)
    ti_d2 = dst_ti.reshape(ROWS, CHUNK)

    hp = _degree_hist(it_s2, it_d2, ti_s2, ti_d2)
    s4 = _scales(hp)
    so_it = s4[0].reshape(-1, 1)
    si_it = s4[1].reshape(-1, 1)
    so_ti = s4[2].reshape(-1, 1)
    si_ti = s4[3].reshape(-1, 1)

    tgt = _ff(input_features.astype(jnp.float32),
              pre_Wi, pre_bi, pre_Wh, pre_bh, pre_Wo, pre_bo)

    edges = (it_s2, it_d2, ti_s2, ti_d2)

    g_item = _rowscale(embed_item, so_it)
    g_tgt = _rowscale(tgt, so_ti)
    p = _agg_pair128(g_item, g_tgt, *edges)
    h1t_s = _conv_post(p[:N_NODE], si_it, c1_W_i2t, c1_b_i2t,
                       relu=True, so=so_ti)
    h1i_s = _conv_post(p[N_NODE:], si_ti, c1_W_t2i, c1_b_t2i,
                       relu=True, so=so_it)

    p = _agg_pair128(h1i_s, h1t_s, *edges)
    h2t_s = _conv_post(p[:N_NODE], si_it, c2_W_i2t, c2_b_i2t,
                       relu=True, so=so_ti)
    h2i_s = _conv_post(p[N_NODE:], si_ti, c2_W_t2i, c2_b_t2i,
                       relu=True, so=so_it)

    p = _agg_pair128(h2i_s, h2t_s, *edges)
    h3_t = _conv_post(p[:N_NODE], si_it, c3_W_i2t, c3_b_i2t, relu=False)
    h3_i = _conv_post(p[N_NODE:], si_ti, c3_W_t2i, c3_b_t2i, relu=False)

    out_t = _ff(h3_t, post_Wi, post_bi, post_Wh, post_bh, post_Wo, post_bo)
    return (out_t, h3_i)


# speedup vs baseline: 1.5966x; 1.5966x over previous
import functools

import jax
import jax.numpy as jnp
from jax import lax
from jax.experimental import pallas as pl
from jax.experimental.pallas import tpu as pltpu
from jax.experimental.pallas import tpu_sc as plsc

N_NODE = 10000
E = 320000
NC = 2
NS = 16
NW = NC * NS
EPW = E // NW
CHUNK = 80
RPT = EPW // CHUNK
ROWS = E // CHUNK
NPT = N_NODE // NS
NZB = NPT // 5

_mesh = plsc.VectorSubcoreMesh(core_axis_name="c", subcore_axis_name="s")
_sc_params = pltpu.CompilerParams(needs_layout_passes=False,
                                  use_tc_tiling_on_sc=False)


@functools.partial(
    pl.kernel, mesh=_mesh, compiler_params=_sc_params,
    out_type=jax.ShapeDtypeStruct((4, NC, N_NODE), jnp.float32),
    scratch_types=[
        pltpu.VMEM((RPT, CHUNK), jnp.int32),
        pltpu.VMEM((CHUNK,), jnp.float32),
        pltpu.VMEM((N_NODE,), jnp.float32),
        pltpu.VMEM_SHARED((N_NODE,), jnp.float32),
        pltpu.VMEM_SHARED((N_NODE,), jnp.float32),
        pltpu.VMEM_SHARED((N_NODE,), jnp.float32),
        pltpu.VMEM_SHARED((N_NODE,), jnp.float32),
        pltpu.SemaphoreType.DMA,
    ],
)
def _degree_hist(src_it, dst_it, src_ti, dst_ti, out, idxbuf, ones, zbuf,
                 a0, a1, a2, a3, sem):
    c = lax.axis_index("c")
    s = lax.axis_index("s")
    wid = s * NC + c
    z16 = jnp.zeros((16,), jnp.float32)
    one16 = jnp.ones((16,), jnp.float32)
    accs = (a0, a1, a2, a3)

    for i in range(CHUNK // 16):
        ones[pl.ds(i * 16, 16)] = one16

    @pl.when(s == 0)
    def _zero():
        def zbody(i, _):
            zbuf[pl.ds(i * 16, 16)] = z16
            return 0
        lax.fori_loop(0, N_NODE // 16, zbody, 0)
        for a in accs:
            pltpu.sync_copy(zbuf, a)
    plsc.subcore_barrier()

    for k, edges in enumerate((src_it, dst_it, src_ti, dst_ti)):
        a = accs[k]
        pltpu.sync_copy(edges.at[pl.ds(wid * RPT, RPT)], idxbuf)

        def fire(j, _):
            pltpu.async_copy(ones, a.at[idxbuf.at[j]], sem, add=True)
            return 0
        lax.fori_loop(0, RPT, fire, 0)

        def drain(j, _):
            pltpu.make_async_copy(ones, a.at[idxbuf.at[0]], sem).wait()
            return 0
        lax.fori_loop(0, RPT, drain, 0)
    plsc.subcore_barrier()

    @pl.when(s == 0)
    def _copyout():
        for k, a in enumerate(accs):
            pltpu.sync_copy(a, zbuf)
            pltpu.sync_copy(zbuf, out.at[k].at[c])


def _zero_acc(zbuf, acc, s, d):
    z16 = jnp.zeros((16,), jnp.float32)

    def zbody(r, _):
        for kcol in range(d // 16):
            zbuf[r, pl.ds(kcol * 16, 16)] = z16
        return 0
    lax.fori_loop(0, NZB, zbody, 0)
    for t in range(5):
        pltpu.sync_copy(zbuf, acc.at[pl.ds(s * NPT + t * NZB, NZB)])


def _copy_out(zbuf, acc, out, s, base):
    for t in range(5):
        off = s * NPT + t * NZB
        pltpu.sync_copy(acc.at[pl.ds(off, NZB)], zbuf)
        pltpu.sync_copy(zbuf, out.at[pl.ds(base + off, NZB)])


IB = 50


def _edge_loop(x, src2, dst2, base, rpt, idx_s, idx_d, rows0, rows1, acc,
               gsem0, gsem1, ssem0, ssem1):
    def wait_gather(rows, gsem):
        pltpu.make_async_copy(x.at[idx_s.at[0]], rows, gsem).wait()

    def wait_scatter(rows, ssem):
        pltpu.make_async_copy(rows, acc.at[idx_d.at[0]], ssem).wait()

    def blk(bi, _):
        pltpu.sync_copy(src2.at[pl.ds(base + bi * IB, IB)], idx_s)
        pltpu.sync_copy(dst2.at[pl.ds(base + bi * IB, IB)], idx_d)
        pltpu.async_copy(x.at[idx_s.at[0]], rows0, gsem0)

        def ebody(k, _):
            j0 = 2 * k

            @pl.when(j0 >= 2)
            def _():
                wait_scatter(rows1, ssem1)
            pltpu.async_copy(x.at[idx_s.at[j0 + 1]], rows1, gsem1)
            wait_gather(rows0, gsem0)
            pltpu.async_copy(rows0, acc.at[idx_d.at[j0]], ssem0, add=True)

            @pl.when(j0 + 2 < IB)
            def _():
                wait_scatter(rows0, ssem0)
                pltpu.async_copy(x.at[idx_s.at[j0 + 2]], rows0, gsem0)
            wait_gather(rows1, gsem1)
            pltpu.async_copy(rows1, acc.at[idx_d.at[j0 + 1]], ssem1, add=True)
            return 0
        lax.fori_loop(0, IB // 2, ebody, 0)
        wait_scatter(rows0, ssem0)
        wait_scatter(rows1, ssem1)
        return 0
    lax.fori_loop(0, rpt // IB, blk, 0)


RPT2 = ROWS // NS


def _make_agg_pair(d):
    @functools.partial(
        pl.kernel, mesh=_mesh, compiler_params=_sc_params,
        out_type=jax.ShapeDtypeStruct((NC * N_NODE, d), jnp.float32),
        scratch_types=[
            pltpu.VMEM((IB, CHUNK), jnp.int32),
            pltpu.VMEM((IB, CHUNK), jnp.int32),
            pltpu.VMEM((CHUNK, d), jnp.float32),
            pltpu.VMEM((CHUNK, d), jnp.float32),
            pltpu.VMEM((NZB, d), jnp.float32),
            pltpu.VMEM_SHARED((N_NODE, d), jnp.float32),
            pltpu.SemaphoreType.DMA,
            pltpu.SemaphoreType.DMA,
            pltpu.SemaphoreType.DMA,
            pltpu.SemaphoreType.DMA,
        ],
    )
    def _agg(xa, xb, sa2, da2, sb2, db2, out, idx_s, idx_d, rows0, rows1,
             zbuf, acc, gsem0, gsem1, ssem0, ssem1):
        c = lax.axis_index("c")
        s = lax.axis_index("s")
        _zero_acc(zbuf, acc, s, d)
        plsc.subcore_barrier()

        @pl.when(c == 0)
        def _ca():
            _edge_loop(xa, sa2, da2, s * RPT2, RPT2, idx_s, idx_d, rows0,
                       rows1, acc, gsem0, gsem1, ssem0, ssem1)

        @pl.when(c == 1)
        def _cb():
            _edge_loop(xb, sb2, db2, s * RPT2, RPT2, idx_s, idx_d, rows0,
                       rows1, acc, gsem0, gsem1, ssem0, ssem1)
        plsc.subcore_barrier()
        _copy_out(zbuf, acc, out, s, c * N_NODE)

    return _agg


_agg_pair128 = _make_agg_pair(128)


BM = 2000


def _dot(a, b):
    return jnp.dot(a, b, preferred_element_type=jnp.float32)


def _scales_body(hp_ref, o_ref):
    deg = jnp.maximum(hp_ref[:, 0] + hp_ref[:, 1], 1.0)
    r = lax.rsqrt(deg)
    r = r * (1.5 - 0.5 * deg * r * r)
    r = r * (1.5 - 0.5 * deg * r * r)
    o_ref[...] = r


def _scales(hp):
    return pl.pallas_call(
        _scales_body,
        out_shape=jax.ShapeDtypeStruct((4, N_NODE), jnp.float32),
    )(hp)


def _rowscale_body(x_ref, s_ref, o_ref):
    o_ref[...] = x_ref[...] * s_ref[...]


def _rowscale(x, sc):
    d = x.shape[-1]
    return pl.pallas_call(
        _rowscale_body,
        grid=(N_NODE // BM,),
        in_specs=[pl.BlockSpec((BM, d), lambda i: (i, 0)),
                  pl.BlockSpec((BM, 1), lambda i: (i, 0))],
        out_specs=pl.BlockSpec((BM, d), lambda i: (i, 0)),
        out_shape=jax.ShapeDtypeStruct((N_NODE, d), jnp.float32),
    )(x, sc)


def _conv_post_body(relu, has_so, p_ref, si_ref, w_ref, b_ref, *rest):
    if has_so:
        so_ref, o_ref = rest
    else:
        (o_ref,) = rest
    v = _dot(p_ref[...] * si_ref[...], w_ref[...]) + b_ref[...]
    if relu:
        v = jnp.maximum(v, 0.0)
    if has_so:
        v = v * so_ref[...]
    o_ref[...] = v


def _conv_post(p, si, w, b, relu, so=None):
    din, dout = w.shape
    has_so = so is not None
    in_specs = [pl.BlockSpec((BM, din), lambda i: (i, 0)),
                pl.BlockSpec((BM, 1), lambda i: (i, 0)),
                pl.BlockSpec((din, dout), lambda i: (0, 0)),
                pl.BlockSpec((1, dout), lambda i: (0, 0))]
    args = [p, si, w, b.reshape(1, -1)]
    if has_so:
        in_specs.append(pl.BlockSpec((BM, 1), lambda i: (i, 0)))
        args.append(so)
    return pl.pallas_call(
        functools.partial(_conv_post_body, relu, has_so),
        grid=(N_NODE // BM,),
        in_specs=in_specs,
        out_specs=pl.BlockSpec((BM, dout), lambda i: (i, 0)),
        out_shape=jax.ShapeDtypeStruct((N_NODE, dout), jnp.float32),
    )(*args)


def _ff_body(x_ref, wi, bi, wh, bh, wo, bo, o_ref):
    h = jnp.maximum(_dot(x_ref[...], wi[...]) + bi[...], 0.0)
    h = jnp.maximum(_dot(h, wh[...]) + bh[...], 0.0)
    o_ref[...] = _dot(h, wo[...]) + bo[...]


def _ff(x, wi, bi, wh, bh, wo, bo):
    m, din = x.shape
    dh = wi.shape[1]
    dout = wo.shape[1]
    return pl.pallas_call(
        _ff_body,
        grid=(m // BM,),
        in_specs=[pl.BlockSpec((BM, din), lambda i: (i, 0)),
                  pl.BlockSpec((din, dh), lambda i: (0, 0)),
                  pl.BlockSpec((1, dh), lambda i: (0, 0)),
                  pl.BlockSpec((dh, dh), lambda i: (0, 0)),
                  pl.BlockSpec((1, dh), lambda i: (0, 0)),
                  pl.BlockSpec((dh, dout), lambda i: (0, 0)),
                  pl.BlockSpec((1, dout), lambda i: (0, 0))],
        out_specs=pl.BlockSpec((BM, dout), lambda i: (i, 0)),
        out_shape=jax.ShapeDtypeStruct((m, dout), jnp.float32),
    )(x, wi, bi.reshape(1, -1), wh, bh.reshape(1, -1), wo, bo.reshape(1, -1))


def kernel(input_features, edge_i2t, edge_t2i, embed_item,
           pre_Wi, pre_bi, pre_Wh, pre_bh, pre_Wo, pre_bo,
           c1_W_i2t, c1_b_i2t, c1_W_t2i, c1_b_t2i,
           c2_W_i2t, c2_b_i2t, c2_W_t2i, c2_b_t2i,
           c3_W_i2t, c3_b_i2t, c3_W_t2i, c3_b_t2i,
           post_Wi, post_bi, post_Wh, post_bh, post_Wo, post_bo):
    src_it = edge_i2t[0].astype(jnp.int32)
    dst_it = edge_i2t[1].astype(jnp.int32)
    src_ti = edge_t2i[0].astype(jnp.int32)
    dst_ti = edge_t2i[1].astype(jnp.int32)

    it_s2 = src_it.reshape(ROWS, CHUNK)
    it_d2 = dst_it.reshape(ROWS, CHUNK)
    ti_s2 = src_ti.reshape(ROWS, CHUNK)
    ti_d2 = dst_ti.reshape(ROWS, CHUNK)

    hp = _degree_hist(it_s2, it_d2, ti_s2, ti_d2)
    s4 = _scales(hp)
    so_it = s4[0].reshape(-1, 1)
    si_it = s4[1].reshape(-1, 1)
    so_ti = s4[2].reshape(-1, 1)
    si_ti = s4[3].reshape(-1, 1)

    tgt = _ff(input_features.astype(jnp.float32),
              pre_Wi, pre_bi, pre_Wh, pre_bh, pre_Wo, pre_bo)

    edges = (it_s2, it_d2, ti_s2, ti_d2)

    g_item = _rowscale(embed_item, so_it)
    g_tgt = _rowscale(tgt, so_ti)
    p = _agg_pair128(g_item, g_tgt, *edges)
    h1t_s = _conv_post(p[:N_NODE], si_it, c1_W_i2t, c1_b_i2t,
                       relu=True, so=so_ti)
    h1i_s = _conv_post(p[N_NODE:], si_ti, c1_W_t2i, c1_b_t2i,
                       relu=True, so=so_it)

    p = _agg_pair128(h1i_s, h1t_s, *edges)
    h2t_s = _conv_post(p[:N_NODE], si_it, c2_W_i2t, c2_b_i2t,
                       relu=True, so=so_ti)
    h2i_s = _conv_post(p[N_NODE:], si_ti, c2_W_t2i, c2_b_t2i,
                       relu=True, so=so_it)

    p = _agg_pair128(h2i_s, h2t_s, *edges)
    h3_t = _conv_post(p[:N_NODE], si_it, c3_W_i2t, c3_b_i2t, relu=False)
    h3_i = _conv_post(p[N_NODE:], si_ti, c3_W_t2i, c3_b_t2i, relu=False)

    out_t = _ff(h3_t, post_Wi, post_bi, post_Wh, post_bh, post_Wo, post_bo)
    return (out_t, h3_i)


# speedup vs baseline: 1.7015x; 1.0657x over previous
import functools

import jax
import jax.numpy as jnp
from jax import lax
from jax.experimental import pallas as pl
from jax.experimental.pallas import tpu as pltpu
from jax.experimental.pallas import tpu_sc as plsc

N_NODE = 10000
E = 320000
NC = 2
NS = 16
NW = NC * NS
EPW = E // NW
CHUNK = 125
RPT = EPW // CHUNK
ROWS = E // CHUNK
NPT = N_NODE // NS
NZB = 25

_mesh = plsc.VectorSubcoreMesh(core_axis_name="c", subcore_axis_name="s")
_sc_params = pltpu.CompilerParams(needs_layout_passes=False,
                                  use_tc_tiling_on_sc=False)


@functools.partial(
    pl.kernel, mesh=_mesh, compiler_params=_sc_params,
    out_type=jax.ShapeDtypeStruct((4, NC, N_NODE), jnp.float32),
    scratch_types=[
        pltpu.VMEM((RPT, CHUNK), jnp.int32),
        pltpu.VMEM((128,), jnp.float32),
        pltpu.VMEM((N_NODE,), jnp.float32),
        pltpu.VMEM_SHARED((N_NODE,), jnp.float32),
        pltpu.VMEM_SHARED((N_NODE,), jnp.float32),
        pltpu.VMEM_SHARED((N_NODE,), jnp.float32),
        pltpu.VMEM_SHARED((N_NODE,), jnp.float32),
        pltpu.SemaphoreType.DMA,
    ],
)
def _degree_hist(src_it, dst_it, src_ti, dst_ti, out, idxbuf, ones, zbuf,
                 a0, a1, a2, a3, sem):
    c = lax.axis_index("c")
    s = lax.axis_index("s")
    wid = s * NC + c
    z16 = jnp.zeros((16,), jnp.float32)
    one16 = jnp.ones((16,), jnp.float32)
    accs = (a0, a1, a2, a3)

    for i in range(128 // 16):
        ones[pl.ds(i * 16, 16)] = one16
    onesc = ones.at[pl.ds(0, CHUNK)]

    @pl.when(s == 0)
    def _zero():
        def zbody(i, _):
            zbuf[pl.ds(i * 16, 16)] = z16
            return 0
        lax.fori_loop(0, N_NODE // 16, zbody, 0)
        for a in accs:
            pltpu.sync_copy(zbuf, a)
    plsc.subcore_barrier()

    for k, edges in enumerate((src_it, dst_it, src_ti, dst_ti)):
        a = accs[k]
        pltpu.sync_copy(edges.at[pl.ds(wid * RPT, RPT)], idxbuf)

        def fire(j, _):
            pltpu.async_copy(onesc, a.at[idxbuf.at[j]], sem, add=True)
            return 0
        lax.fori_loop(0, RPT, fire, 0)

        def drain(j, _):
            pltpu.make_async_copy(onesc, a.at[idxbuf.at[0]], sem).wait()
            return 0
        lax.fori_loop(0, RPT, drain, 0)
    plsc.subcore_barrier()

    @pl.when(s == 0)
    def _copyout():
        for k, a in enumerate(accs):
            pltpu.sync_copy(a, zbuf)
            pltpu.sync_copy(zbuf, out.at[k].at[c])


def _zero_acc(zbuf, acc, s, d):
    z16 = jnp.zeros((16,), jnp.float32)

    def zbody(r, _):
        for kcol in range(d // 16):
            zbuf[r, pl.ds(kcol * 16, 16)] = z16
        return 0
    lax.fori_loop(0, NZB, zbody, 0)

    def cbody(t, _):
        pltpu.sync_copy(zbuf, acc.at[pl.ds(s * NPT + t * NZB, NZB)])
        return 0
    lax.fori_loop(0, NPT // NZB, cbody, 0)


def _copy_out(zbuf, acc, out, s, base):
    def cbody(t, _):
        off = s * NPT + t * NZB
        pltpu.sync_copy(acc.at[pl.ds(off, NZB)], zbuf)
        pltpu.sync_copy(zbuf, out.at[pl.ds(base + off, NZB)])
        return 0
    lax.fori_loop(0, NPT // NZB, cbody, 0)


IB = 40


def _edge_loop(x, src2, dst2, base, rpt, idx_s, idx_d, rows0, rows1, acc,
               gsem0, gsem1, ssem0, ssem1):
    def wait_gather(rows, gsem):
        pltpu.make_async_copy(x.at[idx_s.at[0]], rows, gsem).wait()

    def wait_scatter(rows, ssem):
        pltpu.make_async_copy(rows, acc.at[idx_d.at[0]], ssem).wait()

    def blk(bi, _):
        pltpu.sync_copy(src2.at[pl.ds(base + bi * IB, IB)], idx_s)
        pltpu.sync_copy(dst2.at[pl.ds(base + bi * IB, IB)], idx_d)
        pltpu.async_copy(x.at[idx_s.at[0]], rows0, gsem0)

        def ebody(k, _):
            j0 = 2 * k

            @pl.when(j0 >= 2)
            def _():
                wait_scatter(rows1, ssem1)
            pltpu.async_copy(x.at[idx_s.at[j0 + 1]], rows1, gsem1)
            wait_gather(rows0, gsem0)
            pltpu.async_copy(rows0, acc.at[idx_d.at[j0]], ssem0, add=True)

            @pl.when(j0 + 2 < IB)
            def _():
                wait_scatter(rows0, ssem0)
                pltpu.async_copy(x.at[idx_s.at[j0 + 2]], rows0, gsem0)
            wait_gather(rows1, gsem1)
            pltpu.async_copy(rows1, acc.at[idx_d.at[j0 + 1]], ssem1, add=True)
            return 0
        lax.fori_loop(0, IB // 2, ebody, 0)
        wait_scatter(rows0, ssem0)
        wait_scatter(rows1, ssem1)
        return 0
    lax.fori_loop(0, rpt // IB, blk, 0)


RPT2 = ROWS // NS


def _make_agg_pair(d):
    @functools.partial(
        pl.kernel, mesh=_mesh, compiler_params=_sc_params,
        out_type=jax.ShapeDtypeStruct((NC * N_NODE, d), jnp.float32),
        scratch_types=[
            pltpu.VMEM((IB, CHUNK), jnp.int32),
            pltpu.VMEM((IB, CHUNK), jnp.int32),
            pltpu.VMEM((CHUNK, d), jnp.float32),
            pltpu.VMEM((CHUNK, d), jnp.float32),
            pltpu.VMEM((NZB, d), jnp.float32),
            pltpu.VMEM_SHARED((N_NODE, d), jnp.float32),
            pltpu.SemaphoreType.DMA,
            pltpu.SemaphoreType.DMA,
            pltpu.SemaphoreType.DMA,
            pltpu.SemaphoreType.DMA,
        ],
    )
    def _agg(xa, xb, sa2, da2, sb2, db2, out, idx_s, idx_d, rows0, rows1,
             zbuf, acc, gsem0, gsem1, ssem0, ssem1):
        c = lax.axis_index("c")
        s = lax.axis_index("s")
        _zero_acc(zbuf, acc, s, d)
        plsc.subcore_barrier()

        @pl.when(c == 0)
        def _ca():
            _edge_loop(xa, sa2, da2, s * RPT2, RPT2, idx_s, idx_d, rows0,
                       rows1, acc, gsem0, gsem1, ssem0, ssem1)

        @pl.when(c == 1)
        def _cb():
            _edge_loop(xb, sb2, db2, s * RPT2, RPT2, idx_s, idx_d, rows0,
                       rows1, acc, gsem0, gsem1, ssem0, ssem1)
        plsc.subcore_barrier()
        _copy_out(zbuf, acc, out, s, c * N_NODE)

    return _agg


_agg_pair128 = _make_agg_pair(128)


BM = 2000


def _dot(a, b):
    return jnp.dot(a, b, preferred_element_type=jnp.float32)


def _scales_body(hp_ref, o_ref):
    deg = jnp.maximum(hp_ref[:, 0] + hp_ref[:, 1], 1.0)
    r = lax.rsqrt(deg)
    r = r * (1.5 - 0.5 * deg * r * r)
    r = r * (1.5 - 0.5 * deg * r * r)
    o_ref[...] = r


def _scales(hp):
    return pl.pallas_call(
        _scales_body,
        out_shape=jax.ShapeDtypeStruct((4, N_NODE), jnp.float32),
    )(hp)


def _rowscale_body(x_ref, s_ref, o_ref):
    o_ref[...] = x_ref[...] * s_ref[...]


def _rowscale(x, sc):
    d = x.shape[-1]
    return pl.pallas_call(
        _rowscale_body,
        grid=(N_NODE // BM,),
        in_specs=[pl.BlockSpec((BM, d), lambda i: (i, 0)),
                  pl.BlockSpec((BM, 1), lambda i: (i, 0))],
        out_specs=pl.BlockSpec((BM, d), lambda i: (i, 0)),
        out_shape=jax.ShapeDtypeStruct((N_NODE, d), jnp.float32),
    )(x, sc)


def _conv_post_body(relu, has_so, p_ref, si_ref, w_ref, b_ref, *rest):
    if has_so:
        so_ref, o_ref = rest
    else:
        (o_ref,) = rest
    v = _dot(p_ref[...] * si_ref[...], w_ref[...]) + b_ref[...]
    if relu:
        v = jnp.maximum(v, 0.0)
    if has_so:
        v = v * so_ref[...]
    o_ref[...] = v


def _conv_post(p, si, w, b, relu, so=None):
    din, dout = w.shape
    has_so = so is not None
    in_specs = [pl.BlockSpec((BM, din), lambda i: (i, 0)),
                pl.BlockSpec((BM, 1), lambda i: (i, 0)),
                pl.BlockSpec((din, dout), lambda i: (0, 0)),
                pl.BlockSpec((1, dout), lambda i: (0, 0))]
    args = [p, si, w, b.reshape(1, -1)]
    if has_so:
        in_specs.append(pl.BlockSpec((BM, 1), lambda i: (i, 0)))
        args.append(so)
    return pl.pallas_call(
        functools.partial(_conv_post_body, relu, has_so),
        grid=(N_NODE // BM,),
        in_specs=in_specs,
        out_specs=pl.BlockSpec((BM, dout), lambda i: (i, 0)),
        out_shape=jax.ShapeDtypeStruct((N_NODE, dout), jnp.float32),
    )(*args)


def _ff_body(x_ref, wi, bi, wh, bh, wo, bo, o_ref):
    h = jnp.maximum(_dot(x_ref[...], wi[...]) + bi[...], 0.0)
    h = jnp.maximum(_dot(h, wh[...]) + bh[...], 0.0)
    o_ref[...] = _dot(h, wo[...]) + bo[...]


def _ff(x, wi, bi, wh, bh, wo, bo):
    m, din = x.shape
    dh = wi.shape[1]
    dout = wo.shape[1]
    return pl.pallas_call(
        _ff_body,
        grid=(m // BM,),
        in_specs=[pl.BlockSpec((BM, din), lambda i: (i, 0)),
                  pl.BlockSpec((din, dh), lambda i: (0, 0)),
                  pl.BlockSpec((1, dh), lambda i: (0, 0)),
                  pl.BlockSpec((dh, dh), lambda i: (0, 0)),
                  pl.BlockSpec((1, dh), lambda i: (0, 0)),
                  pl.BlockSpec((dh, dout), lambda i: (0, 0)),
                  pl.BlockSpec((1, dout), lambda i: (0, 0))],
        out_specs=pl.BlockSpec((BM, dout), lambda i: (i, 0)),
        out_shape=jax.ShapeDtypeStruct((m, dout), jnp.float32),
    )(x, wi, bi.reshape(1, -1), wh, bh.reshape(1, -1), wo, bo.reshape(1, -1))


def kernel(input_features, edge_i2t, edge_t2i, embed_item,
           pre_Wi, pre_bi, pre_Wh, pre_bh, pre_Wo, pre_bo,
           c1_W_i2t, c1_b_i2t, c1_W_t2i, c1_b_t2i,
           c2_W_i2t, c2_b_i2t, c2_W_t2i, c2_b_t2i,
           c3_W_i2t, c3_b_i2t, c3_W_t2i, c3_b_t2i,
           post_Wi, post_bi, post_Wh, post_bh, post_Wo, post_bo):
    src_it = edge_i2t[0].astype(jnp.int32)
    dst_it = edge_i2t[1].astype(jnp.int32)
    src_ti = edge_t2i[0].astype(jnp.int32)
    dst_ti = edge_t2i[1].astype(jnp.int32)

    it_s2 = src_it.reshape(ROWS, CHUNK)
    it_d2 = dst_it.reshape(ROWS, CHUNK)
    ti_s2 = src_ti.reshape(ROWS, CHUNK)
    ti_d2 = dst_ti.reshape(ROWS, CHUNK)

    hp = _degree_hist(it_s2, it_d2, ti_s2, ti_d2)
    s4 = _scales(hp)
    so_it = s4[0].reshape(-1, 1)
    si_it = s4[1].reshape(-1, 1)
    so_ti = s4[2].reshape(-1, 1)
    si_ti = s4[3].reshape(-1, 1)

    tgt = _ff(input_features.astype(jnp.float32),
              pre_Wi, pre_bi, pre_Wh, pre_bh, pre_Wo, pre_bo)

    edges = (it_s2, it_d2, ti_s2, ti_d2)

    g_item = _rowscale(embed_item, so_it)
    g_tgt = _rowscale(tgt, so_ti)
    p = _agg_pair128(g_item, g_tgt, *edges)
    h1t_s = _conv_post(p[:N_NODE], si_it, c1_W_i2t, c1_b_i2t,
                       relu=True, so=so_ti)
    h1i_s = _conv_post(p[N_NODE:], si_ti, c1_W_t2i, c1_b_t2i,
                       relu=True, so=so_it)

    p = _agg_pair128(h1i_s, h1t_s, *edges)
    h2t_s = _conv_post(p[:N_NODE], si_it, c2_W_i2t, c2_b_i2t,
                       relu=True, so=so_ti)
    h2i_s = _conv_post(p[N_NODE:], si_ti, c2_W_t2i, c2_b_t2i,
                       relu=True, so=so_it)

    p = _agg_pair128(h2i_s, h2t_s, *edges)
    h3_t = _conv_post(p[:N_NODE], si_it, c3_W_i2t, c3_b_i2t, relu=False)
    h3_i = _conv_post(p[N_NODE:], si_ti, c3_W_t2i, c3_b_t2i, relu=False)

    out_t = _ff(h3_t, post_Wi, post_bi, post_Wh, post_bh, post_Wo, post_bo)
    return (out_t, h3_i)


# speedup vs baseline: 1.7278x; 1.0155x over previous
import functools

import jax
import jax.numpy as jnp
from jax import lax
from jax.experimental import pallas as pl
from jax.experimental.pallas import tpu as pltpu
from jax.experimental.pallas import tpu_sc as plsc

N_NODE = 10000
E = 320000
NC = 2
NS = 16
NW = NC * NS
EPW = E // NW
CHUNK = 125
RPT = EPW // CHUNK
ROWS = E // CHUNK
NPT = N_NODE // NS
NZB = 25

_mesh = plsc.VectorSubcoreMesh(core_axis_name="c", subcore_axis_name="s")
_sc_params = pltpu.CompilerParams(needs_layout_passes=False,
                                  use_tc_tiling_on_sc=False)


@functools.partial(
    pl.kernel, mesh=_mesh, compiler_params=_sc_params,
    out_type=jax.ShapeDtypeStruct((4, NC, N_NODE), jnp.float32),
    scratch_types=[
        pltpu.VMEM((RPT, CHUNK), jnp.int32),
        pltpu.VMEM((128,), jnp.float32),
        pltpu.VMEM((N_NODE,), jnp.float32),
        pltpu.VMEM_SHARED((N_NODE,), jnp.float32),
        pltpu.VMEM_SHARED((N_NODE,), jnp.float32),
        pltpu.VMEM_SHARED((N_NODE,), jnp.float32),
        pltpu.VMEM_SHARED((N_NODE,), jnp.float32),
        pltpu.SemaphoreType.DMA,
    ],
)
def _degree_hist(src_it, dst_it, src_ti, dst_ti, out, idxbuf, ones, zbuf,
                 a0, a1, a2, a3, sem):
    c = lax.axis_index("c")
    s = lax.axis_index("s")
    wid = s * NC + c
    z16 = jnp.zeros((16,), jnp.float32)
    one16 = jnp.ones((16,), jnp.float32)
    accs = (a0, a1, a2, a3)

    for i in range(128 // 16):
        ones[pl.ds(i * 16, 16)] = one16
    onesc = ones.at[pl.ds(0, CHUNK)]

    @pl.when(s == 0)
    def _zero():
        def zbody(i, _):
            zbuf[pl.ds(i * 16, 16)] = z16
            return 0
        lax.fori_loop(0, N_NODE // 16, zbody, 0)
        for a in accs:
            pltpu.sync_copy(zbuf, a)
    plsc.subcore_barrier()

    for k, edges in enumerate((src_it, dst_it, src_ti, dst_ti)):
        a = accs[k]
        pltpu.sync_copy(edges.at[pl.ds(wid * RPT, RPT)], idxbuf)

        def fire(j, _):
            pltpu.async_copy(onesc, a.at[idxbuf.at[j]], sem, add=True)
            return 0
        lax.fori_loop(0, RPT, fire, 0)

        def drain(j, _):
            pltpu.make_async_copy(onesc, a.at[idxbuf.at[0]], sem).wait()
            return 0
        lax.fori_loop(0, RPT, drain, 0)
    plsc.subcore_barrier()

    @pl.when(s == 0)
    def _copyout():
        for k, a in enumerate(accs):
            pltpu.sync_copy(a, zbuf)
            pltpu.sync_copy(zbuf, out.at[k].at[c])


def _zero_acc(zbuf, acc, s, d, sem):
    z16 = jnp.zeros((16,), jnp.float32)

    def zbody(r, _):
        for kcol in range(d // 16):
            zbuf[r, pl.ds(kcol * 16, 16)] = z16
        return 0
    lax.fori_loop(0, NZB, zbody, 0)

    def fire(t, _):
        pltpu.async_copy(zbuf, acc.at[pl.ds(s * NPT + t * NZB, NZB)], sem)
        return 0
    lax.fori_loop(0, NPT // NZB, fire, 0)

    def drain(t, _):
        pltpu.make_async_copy(zbuf, acc.at[pl.ds(s * NPT, NZB)], sem).wait()
        return 0
    lax.fori_loop(0, NPT // NZB, drain, 0)


def _copy_out(zbuf, rbuf, acc, out, s, base, sem0, sem1):
    nb = NPT // NZB

    def rd(t, buf, sem):
        pltpu.async_copy(acc.at[pl.ds(s * NPT + t * NZB, NZB)], buf, sem)

    def wrt(t, buf, sem):
        pltpu.make_async_copy(acc.at[pl.ds(s * NPT, NZB)], buf, sem).wait()
        pltpu.sync_copy(buf, out.at[pl.ds(base + s * NPT + t * NZB, NZB)])

    rd(0, zbuf, sem0)

    def body(k, _):
        t0 = 2 * k

        @pl.when(t0 + 1 < nb)
        def _():
            rd(t0 + 1, rbuf, sem1)
        wrt(t0, zbuf, sem0)

        @pl.when(t0 + 2 < nb)
        def _():
            rd(t0 + 2, zbuf, sem0)

        @pl.when(t0 + 1 < nb)
        def _():
            wrt(t0 + 1, rbuf, sem1)
        return 0
    lax.fori_loop(0, (nb + 1) // 2, body, 0)


IB = 40


def _edge_loop(x, src2, dst2, base, rpt, idx_s, idx_d, rows0, rows1, acc,
               gsem0, gsem1, ssem0, ssem1):
    def wait_gather(rows, gsem):
        pltpu.make_async_copy(x.at[idx_s.at[0]], rows, gsem).wait()

    def wait_scatter(rows, ssem):
        pltpu.make_async_copy(rows, acc.at[idx_d.at[0]], ssem).wait()

    def blk(bi, _):
        pltpu.sync_copy(src2.at[pl.ds(base + bi * IB, IB)], idx_s)
        pltpu.sync_copy(dst2.at[pl.ds(base + bi * IB, IB)], idx_d)
        pltpu.async_copy(x.at[idx_s.at[0]], rows0, gsem0)

        def ebody(k, _):
            j0 = 2 * k

            @pl.when(j0 >= 2)
            def _():
                wait_scatter(rows1, ssem1)
            pltpu.async_copy(x.at[idx_s.at[j0 + 1]], rows1, gsem1)
            wait_gather(rows0, gsem0)
            pltpu.async_copy(rows0, acc.at[idx_d.at[j0]], ssem0, add=True)

            @pl.when(j0 + 2 < IB)
            def _():
                wait_scatter(rows0, ssem0)
                pltpu.async_copy(x.at[idx_s.at[j0 + 2]], rows0, gsem0)
            wait_gather(rows1, gsem1)
            pltpu.async_copy(rows1, acc.at[idx_d.at[j0 + 1]], ssem1, add=True)
            return 0
        lax.fori_loop(0, IB // 2, ebody, 0)
        wait_scatter(rows0, ssem0)
        wait_scatter(rows1, ssem1)
        return 0
    lax.fori_loop(0, rpt // IB, blk, 0)


RPT2 = ROWS // NS


def _make_agg_pair(d):
    @functools.partial(
        pl.kernel, mesh=_mesh, compiler_params=_sc_params,
        out_type=jax.ShapeDtypeStruct((NC * N_NODE, d), jnp.float32),
        scratch_types=[
            pltpu.VMEM((IB, CHUNK), jnp.int32),
            pltpu.VMEM((IB, CHUNK), jnp.int32),
            pltpu.VMEM((CHUNK, d), jnp.float32),
            pltpu.VMEM((CHUNK, d), jnp.float32),
            pltpu.VMEM((NZB, d), jnp.float32),
            pltpu.VMEM_SHARED((N_NODE, d), jnp.float32),
            pltpu.SemaphoreType.DMA,
            pltpu.SemaphoreType.DMA,
            pltpu.SemaphoreType.DMA,
            pltpu.SemaphoreType.DMA,
        ],
    )
    def _agg(xa, xb, sa2, da2, sb2, db2, out, idx_s, idx_d, rows0, rows1,
             zbuf, acc, gsem0, gsem1, ssem0, ssem1):
        c = lax.axis_index("c")
        s = lax.axis_index("s")
        _zero_acc(zbuf, acc, s, d, gsem0)
        plsc.subcore_barrier()

        @pl.when(c == 0)
        def _ca():
            _edge_loop(xa, sa2, da2, s * RPT2, RPT2, idx_s, idx_d, rows0,
                       rows1, acc, gsem0, gsem1, ssem0, ssem1)

        @pl.when(c == 1)
        def _cb():
            _edge_loop(xb, sb2, db2, s * RPT2, RPT2, idx_s, idx_d, rows0,
                       rows1, acc, gsem0, gsem1, ssem0, ssem1)
        plsc.subcore_barrier()
        _copy_out(zbuf, rows0.at[pl.ds(0, NZB)], acc, out, s, c * N_NODE,
                  gsem0, gsem1)

    return _agg


_agg_pair128 = _make_agg_pair(128)


BM = 2000


def _dot(a, b):
    return jnp.dot(a, b, preferred_element_type=jnp.float32)


def _scales_body(hp_ref, o_ref):
    deg = jnp.maximum(hp_ref[:, 0] + hp_ref[:, 1], 1.0)
    r = lax.rsqrt(deg)
    r = r * (1.5 - 0.5 * deg * r * r)
    r = r * (1.5 - 0.5 * deg * r * r)
    o_ref[...] = r


def _scales(hp):
    return pl.pallas_call(
        _scales_body,
        out_shape=jax.ShapeDtypeStruct((4, N_NODE), jnp.float32),
    )(hp)


def _rowscale_body(x_ref, s_ref, o_ref):
    o_ref[...] = x_ref[...] * s_ref[...]


def _rowscale(x, sc):
    d = x.shape[-1]
    return pl.pallas_call(
        _rowscale_body,
        grid=(N_NODE // BM,),
        in_specs=[pl.BlockSpec((BM, d), lambda i: (i, 0)),
                  pl.BlockSpec((BM, 1), lambda i: (i, 0))],
        out_specs=pl.BlockSpec((BM, d), lambda i: (i, 0)),
        out_shape=jax.ShapeDtypeStruct((N_NODE, d), jnp.float32),
    )(x, sc)


def _conv_post_body(relu, has_so, p_ref, si_ref, w_ref, b_ref, *rest):
    if has_so:
        so_ref, o_ref = rest
    else:
        (o_ref,) = rest
    v = _dot(p_ref[...] * si_ref[...], w_ref[...]) + b_ref[...]
    if relu:
        v = jnp.maximum(v, 0.0)
    if has_so:
        v = v * so_ref[...]
    o_ref[...] = v


def _conv_post(p, si, w, b, relu, so=None):
    din, dout = w.shape
    has_so = so is not None
    in_specs = [pl.BlockSpec((BM, din), lambda i: (i, 0)),
                pl.BlockSpec((BM, 1), lambda i: (i, 0)),
                pl.BlockSpec((din, dout), lambda i: (0, 0)),
                pl.BlockSpec((1, dout), lambda i: (0, 0))]
    args = [p, si, w, b.reshape(1, -1)]
    if has_so:
        in_specs.append(pl.BlockSpec((BM, 1), lambda i: (i, 0)))
        args.append(so)
    return pl.pallas_call(
        functools.partial(_conv_post_body, relu, has_so),
        grid=(N_NODE // BM,),
        in_specs=in_specs,
        out_specs=pl.BlockSpec((BM, dout), lambda i: (i, 0)),
        out_shape=jax.ShapeDtypeStruct((N_NODE, dout), jnp.float32),
    )(*args)


def _ff_body(has_so, x_ref, wi, bi, wh, bh, wo, bo, *rest):
    if has_so:
        so_ref, o_ref = rest
    else:
        (o_ref,) = rest
    h = jnp.maximum(_dot(x_ref[...], wi[...]) + bi[...], 0.0)
    h = jnp.maximum(_dot(h, wh[...]) + bh[...], 0.0)
    v = _dot(h, wo[...]) + bo[...]
    if has_so:
        v = v * so_ref[...]
    o_ref[...] = v


def _ff(x, wi, bi, wh, bh, wo, bo, so=None):
    m, din = x.shape
    dh = wi.shape[1]
    dout = wo.shape[1]
    has_so = so is not None
    in_specs = [pl.BlockSpec((BM, din), lambda i: (i, 0)),
                pl.BlockSpec((din, dh), lambda i: (0, 0)),
                pl.BlockSpec((1, dh), lambda i: (0, 0)),
                pl.BlockSpec((dh, dh), lambda i: (0, 0)),
                pl.BlockSpec((1, dh), lambda i: (0, 0)),
                pl.BlockSpec((dh, dout), lambda i: (0, 0)),
                pl.BlockSpec((1, dout), lambda i: (0, 0))]
    args = [x, wi, bi.reshape(1, -1), wh, bh.reshape(1, -1), wo,
            bo.reshape(1, -1)]
    if has_so:
        in_specs.append(pl.BlockSpec((BM, 1), lambda i: (i, 0)))
        args.append(so)
    return pl.pallas_call(
        functools.partial(_ff_body, has_so),
        grid=(m // BM,),
        in_specs=in_specs,
        out_specs=pl.BlockSpec((BM, dout), lambda i: (i, 0)),
        out_shape=jax.ShapeDtypeStruct((m, dout), jnp.float32),
    )(*args)


def kernel(input_features, edge_i2t, edge_t2i, embed_item,
           pre_Wi, pre_bi, pre_Wh, pre_bh, pre_Wo, pre_bo,
           c1_W_i2t, c1_b_i2t, c1_W_t2i, c1_b_t2i,
           c2_W_i2t, c2_b_i2t, c2_W_t2i, c2_b_t2i,
           c3_W_i2t, c3_b_i2t, c3_W_t2i, c3_b_t2i,
           post_Wi, post_bi, post_Wh, post_bh, post_Wo, post_bo):
    src_it = edge_i2t[0].astype(jnp.int32)
    dst_it = edge_i2t[1].astype(jnp.int32)
    src_ti = edge_t2i[0].astype(jnp.int32)
    dst_ti = edge_t2i[1].astype(jnp.int32)

    it_s2 = src_it.reshape(ROWS, CHUNK)
    it_d2 = dst_it.reshape(ROWS, CHUNK)
    ti_s2 = src_ti.reshape(ROWS, CHUNK)
    ti_d2 = dst_ti.reshape(ROWS, CHUNK)

    hp = _degree_hist(it_s2, it_d2, ti_s2, ti_d2)
    s4 = _scales(hp)
    so_it = s4[0].reshape(-1, 1)
    si_it = s4[1].reshape(-1, 1)
    so_ti = s4[2].reshape(-1, 1)
    si_ti = s4[3].reshape(-1, 1)

    edges = (it_s2, it_d2, ti_s2, ti_d2)

    g_tgt = _ff(input_features.astype(jnp.float32),
                pre_Wi, pre_bi, pre_Wh, pre_bh, pre_Wo, pre_bo,
                so=so_ti)
    g_item = _rowscale(embed_item, so_it)
    p = _agg_pair128(g_item, g_tgt, *edges)
    h1t_s = _conv_post(p[:N_NODE], si_it, c1_W_i2t, c1_b_i2t,
                       relu=True, so=so_ti)
    h1i_s = _conv_post(p[N_NODE:], si_ti, c1_W_t2i, c1_b_t2i,
                       relu=True, so=so_it)

    p = _agg_pair128(h1i_s, h1t_s, *edges)
    h2t_s = _conv_post(p[:N_NODE], si_it, c2_W_i2t, c2_b_i2t,
                       relu=True, so=so_ti)
    h2i_s = _conv_post(p[N_NODE:], si_ti, c2_W_t2i, c2_b_t2i,
                       relu=True, so=so_it)

    p = _agg_pair128(h2i_s, h2t_s, *edges)
    h3_t = _conv_post(p[:N_NODE], si_it, c3_W_i2t, c3_b_i2t, relu=False)
    h3_i = _conv_post(p[N_NODE:], si_ti, c3_W_t2i, c3_b_t2i, relu=False)

    out_t = _ff(h3_t, post_Wi, post_bi, post_Wh, post_bh, post_Wo, post_bo)
    return (out_t, h3_i)
